# Initial kernel scaffold; baseline (speedup 1.0000x reference)
import jax
import jax.numpy as jnp
from jax.experimental import pallas as pl


def kernel(x, edge_index0, edge_index1, W1, b1, W2a, b2a, W2b, b2b):
    raise NotImplementedError("write your pallas kernel here")


# speedup vs baseline: 4.4744x; 4.4744x over previous
import functools

import jax
import jax.numpy as jnp
from jax import lax
from jax.experimental import pallas as pl
from jax.experimental.pallas import tpu as pltpu
from jax.experimental.pallas import tpu_sc as plsc

N0 = 10000
N1 = 5000
N2 = 2000
E0 = 320000
E1 = 160000
D = 128
H = 256
O = 64

NC = 2
NS = 16
NW = NC * NS
L = 16

N1P = 5120
N2P = 2048
DEGW = 16
ZR = 16


@functools.lru_cache(maxsize=None)
def _make_segsum(n_tgt_pad: int, e_total: int, chunk: int):
    per_w = e_total // NW
    iters = per_w // chunk
    assert per_w % chunk == 0 and chunk % 8 == 0 and chunk <= 128
    rows_per_sub = n_tgt_pad // NS
    assert rows_per_sub % ZR == 0

    mesh = plsc.VectorSubcoreMesh(core_axis_name="c", subcore_axis_name="s")

    @functools.partial(
        pl.kernel,
        mesh=mesh,
        out_type=[
            jax.ShapeDtypeStruct((NC, n_tgt_pad, D), jnp.float32),
            jax.ShapeDtypeStruct((NC, n_tgt_pad, DEGW), jnp.float32),
        ],
        scratch_types=[
            pltpu.VMEM((chunk,), jnp.int32),
            pltpu.VMEM((chunk,), jnp.int32),
            pltpu.VMEM((chunk, D), jnp.float32),
            pltpu.VMEM((chunk, DEGW), jnp.float32),
            pltpu.VMEM((ZR, D), jnp.float32),
            pltpu.VMEM((ZR, DEGW), jnp.float32),
            pltpu.VMEM_SHARED((n_tgt_pad, D), jnp.float32),
            pltpu.VMEM_SHARED((n_tgt_pad, DEGW), jnp.float32),
            pltpu.SemaphoreType.DMA,
        ],
    )
    def k(table, src_i, dst_i, acc_out, deg_out,
          src_v, dst_v, rows_v, ones_v, zrow_v, zdeg_v, acc_sh, deg_sh, sem):
        cid = lax.axis_index("c")
        sid = lax.axis_index("s")

        z16 = jnp.zeros((L,), jnp.float32)
        o16 = jnp.ones((L,), jnp.float32)

        def fill_zrow(i, _):
            r = i // (D // L)
            c = (i % (D // L)) * L
            zrow_v[r, pl.ds(c, L)] = z16
            return 0
        lax.fori_loop(0, ZR * (D // L), fill_zrow, 0)

        def fill_zdeg(i, _):
            zdeg_v[i, pl.ds(0, L)] = z16
            return 0
        lax.fori_loop(0, ZR, fill_zdeg, 0)

        def fill_ones(i, _):
            ones_v[i, pl.ds(0, L)] = o16
            return 0
        lax.fori_loop(0, chunk, fill_ones, 0)

        base_r = sid * rows_per_sub

        def zero_acc(i, _):
            pltpu.sync_copy(zrow_v, acc_sh.at[pl.ds(base_r + i * ZR, ZR)])
            pltpu.sync_copy(zdeg_v, deg_sh.at[pl.ds(base_r + i * ZR, ZR)])
            return 0
        lax.fori_loop(0, rows_per_sub // ZR, zero_acc, 0)

        plsc.subcore_barrier()

        wid = sid * NC + cid
        ebase = wid * per_w

        def body(t, _):
            off = ebase + t * chunk
            pltpu.sync_copy(src_i.at[pl.ds(off, chunk)], src_v)
            pltpu.sync_copy(dst_i.at[pl.ds(off, chunk)], dst_v)
            pltpu.async_copy(table.at[src_v], rows_v, sem).wait()
            pltpu.sync_copy(rows_v, acc_sh.at[dst_v], add=True)
            pltpu.sync_copy(ones_v, deg_sh.at[dst_v], add=True)
            return 0
        lax.fori_loop(0, iters, body, 0)

        plsc.subcore_barrier()

        pltpu.sync_copy(acc_sh.at[pl.ds(base_r, rows_per_sub)],
                        acc_out.at[cid, pl.ds(base_r, rows_per_sub)])
        pltpu.sync_copy(deg_sh.at[pl.ds(base_r, rows_per_sub)],
                        deg_out.at[cid, pl.ds(base_r, rows_per_sub)])

    return k


def _mlp1_body(a0, a1, d0, d1, w, b, out):
    deg = d0[:, 0:1] + d1[:, 0:1]
    a = (a0[...] + a1[...]) / jnp.maximum(deg, 1.0)
    y = jnp.dot(a, w[...], preferred_element_type=jnp.float32) + b[...]
    n = jnp.sqrt(jnp.sum(y * y, axis=-1, keepdims=True))
    out[...] = y / jnp.maximum(n, 1e-12)


def _mlp1(acc, deg, W1, b1):
    BR = 640
    grid = N1P // BR
    return pl.pallas_call(
        _mlp1_body,
        grid=(grid,),
        in_specs=[
            pl.BlockSpec((BR, D), lambda i: (i, 0)),
            pl.BlockSpec((BR, D), lambda i: (i, 0)),
            pl.BlockSpec((BR, DEGW), lambda i: (i, 0)),
            pl.BlockSpec((BR, DEGW), lambda i: (i, 0)),
            pl.BlockSpec((D, D), lambda i: (0, 0)),
            pl.BlockSpec((1, D), lambda i: (0, 0)),
        ],
        out_specs=pl.BlockSpec((BR, D), lambda i: (i, 0)),
        out_shape=jax.ShapeDtypeStruct((N1P, D), jnp.float32),
    )(acc[0], acc[1], deg[0], deg[1], W1, b1)


def _mlp2_body(a0, a1, d0, d1, wa, ba, wb, bb, out):
    deg = d0[:, 0:1] + d1[:, 0:1]
    a = (a0[...] + a1[...]) / jnp.maximum(deg, 1.0)
    y = jnp.dot(a, wa[...], preferred_element_type=jnp.float32) + ba[...]
    y = jnp.maximum(y, 0.0)
    z = jnp.dot(y, wb[...], preferred_element_type=jnp.float32) + bb[...]
    n = jnp.sqrt(jnp.sum(z * z, axis=-1, keepdims=True))
    out[...] = z / jnp.maximum(n, 1e-12)


def _mlp2(acc, deg, W2a, b2a, W2b, b2b):
    BR = 512
    grid = N2P // BR
    return pl.pallas_call(
        _mlp2_body,
        grid=(grid,),
        in_specs=[
            pl.BlockSpec((BR, D), lambda i: (i, 0)),
            pl.BlockSpec((BR, D), lambda i: (i, 0)),
            pl.BlockSpec((BR, DEGW), lambda i: (i, 0)),
            pl.BlockSpec((BR, DEGW), lambda i: (i, 0)),
            pl.BlockSpec((D, H), lambda i: (0, 0)),
            pl.BlockSpec((1, H), lambda i: (0, 0)),
            pl.BlockSpec((H, O), lambda i: (0, 0)),
            pl.BlockSpec((1, O), lambda i: (0, 0)),
        ],
        out_specs=pl.BlockSpec((BR, O), lambda i: (i, 0)),
        out_shape=jax.ShapeDtypeStruct((N2P, O), jnp.float32),
    )(acc[0], acc[1], deg[0], deg[1], W2a, b2a, W2b, b2b)


def kernel(x, edge_index0, edge_index1, W1, b1, W2a, b2a, W2b, b2b):
    src0 = edge_index0[0].astype(jnp.int32)
    dst0 = edge_index0[1].astype(jnp.int32)
    src1 = edge_index1[0].astype(jnp.int32)
    dst1 = edge_index1[1].astype(jnp.int32)

    acc0, deg0 = _make_segsum(N1P, E0, 80)(x, src0, dst0)
    h = _mlp1(acc0, deg0, W1, b1.reshape(1, D))
    acc1, deg1 = _make_segsum(N2P, E1, 40)(h, src1, dst1)
    out = _mlp2(acc1, deg1, W2a, b2a.reshape(1, H), W2b, b2b.reshape(1, O))
    return out[:N2]
